# baseline (speedup 1.0000x reference)
import jax
import jax.numpy as jnp
from jax.experimental import pallas as pl
from jax.experimental.pallas import tpu as pltpu

_BM = 200


def _gcn_kernel(x_ref, a0_ref, a1_ref, gc1_w_ref, gc1_b_ref, gc2_b_ref,
                lin1_w_ref, lin1_b_ref, w2c_ref, lin2_b_ref,
                out_ref, sab_s, sc_s, misc_s):
    j = pl.program_id(0)
    i = pl.program_id(1)
    bm = a0_ref.shape[0]
    nhid = gc1_w_ref.shape[1]
    rows = pl.ds(2 * i * bm, 2 * bm)

    @pl.when(jnp.logical_and(j == 0, i == 0))
    def _proj():
        xx = x_ref[...]
        sA = jnp.dot(xx, gc1_w_ref[...], preferred_element_type=jnp.float32)
        l1 = jnp.dot(xx, lin1_w_ref[...], preferred_element_type=jnp.float32)
        l1 = l1 + lin1_b_ref[...]
        sBu = jnp.dot(l1, w2c_ref[...], preferred_element_type=jnp.float32)
        sab_s[...] = jnp.concatenate([sA, sBu[:, 0:8]],
                                     axis=1).astype(jnp.bfloat16)
        misc_s[:, 0:8] = sBu[:, 8:16] + lin2_b_ref[...]

    @pl.when(j == 0)
    def _pass1():
        h0 = jnp.dot(a0_ref[...].astype(jnp.bfloat16), sab_s[...],
                     preferred_element_type=jnp.float32)
        h1 = jnp.dot(a1_ref[...].astype(jnp.bfloat16), sab_s[...],
                     preferred_element_type=jnp.float32)
        h = jnp.concatenate([h0, h1], axis=0)
        r1 = jnp.maximum(h[:, :nhid] + gc1_b_ref[...], 0.0)
        scv = jnp.dot(r1, w2c_ref[...], preferred_element_type=jnp.float32)
        sc_s[rows, :] = scv[:, 0:8].astype(jnp.bfloat16)
        misc_s[rows, 8:16] = h[:, nhid:] + gc2_b_ref[...]
        misc_s[rows, 16:24] = scv[:, 8:16] + misc_s[rows, 0:8]

    @pl.when(j == 1)
    def _pass2():
        hc0 = jnp.dot(a0_ref[...].astype(jnp.bfloat16), sc_s[...],
                      preferred_element_type=jnp.float32)
        hc1 = jnp.dot(a1_ref[...].astype(jnp.bfloat16), sc_s[...],
                      preferred_element_type=jnp.float32)
        hc = jnp.concatenate([hc0, hc1], axis=0)
        out_ref[...] = (jnp.maximum(hc + misc_s[rows, 8:16], 0.0)
                        + misc_s[rows, 16:24])


@jax.jit
def kernel(x, adj, gc1_w, gc1_b, gc2_w, gc2_b,
           lin1_w, lin1_b, lin2_w, lin2_b):
    n, nfeat = x.shape
    nhid = gc1_w.shape[1]
    ncls = gc2_w.shape[1]
    w2c = jnp.concatenate([gc2_w, lin2_w], axis=1)

    full = lambda r, c: pl.BlockSpec((r, c), lambda j, i: (0, 0))

    out = pl.pallas_call(
        _gcn_kernel,
        grid=(2, n // (2 * _BM)),
        in_specs=[
            full(n, nfeat),
            pl.BlockSpec((_BM, n), lambda j, i: (2 * i, 0)),
            pl.BlockSpec((_BM, n), lambda j, i: (2 * i + 1, 0)),
            full(nfeat, nhid),
            full(1, nhid),
            full(1, ncls),
            full(nfeat, nhid),
            full(1, nhid),
            full(nhid, 2 * ncls),
            full(1, ncls),
        ],
        out_specs=pl.BlockSpec((2 * _BM, ncls), lambda j, i: (i, 0)),
        out_shape=jax.ShapeDtypeStruct((n, ncls), jnp.float32),
        scratch_shapes=[
            pltpu.VMEM((n, nhid + ncls), jnp.bfloat16),
            pltpu.VMEM((n, ncls), jnp.bfloat16),
            pltpu.VMEM((n, 3 * ncls), jnp.float32),
        ],
        compiler_params=pltpu.CompilerParams(
            dimension_semantics=("arbitrary", "arbitrary"),
        ),
    )(x, adj, adj, gc1_w, gc1_b.reshape(1, nhid), gc2_b.reshape(1, ncls),
      lin1_w, lin1_b.reshape(1, nhid), w2c, lin2_b.reshape(1, ncls))
    return out


# speedup vs baseline: 1.0063x; 1.0063x over previous
import jax
import jax.numpy as jnp
from jax.experimental import pallas as pl
from jax.experimental.pallas import tpu as pltpu

_BM = 80
_NWIN = 5


def _gcn_kernel(x_ref, a0_ref, a1_ref, a2_ref, a3_ref, a4_ref,
                gc1_w_ref, gc1_b_ref, gc2_b_ref,
                lin1_w_ref, lin1_b_ref, w2c_ref, lin2_b_ref,
                out_ref, sab_s, sc_s, misc_s):
    j = pl.program_id(0)
    i = pl.program_id(1)
    bm = a0_ref.shape[0]
    nhid = gc1_w_ref.shape[1]
    rows = pl.ds(_NWIN * i * bm, _NWIN * bm)

    @pl.when(jnp.logical_and(j == 0, i == 0))
    def _proj():
        xx = x_ref[...]
        sA = jnp.dot(xx, gc1_w_ref[...], preferred_element_type=jnp.float32)
        l1 = jnp.dot(xx, lin1_w_ref[...], preferred_element_type=jnp.float32)
        l1 = l1 + lin1_b_ref[...]
        sBu = jnp.dot(l1, w2c_ref[...], preferred_element_type=jnp.float32)
        sab_s[...] = jnp.concatenate([sA, sBu[:, 0:8]],
                                     axis=1).astype(jnp.bfloat16)
        misc_s[:, 0:8] = sBu[:, 8:16] + lin2_b_ref[...]

    @pl.when(j == 0)
    def _pass1():
        h = jnp.concatenate(
            [jnp.dot(a_ref[...].astype(jnp.bfloat16), sab_s[...],
                     preferred_element_type=jnp.float32)
             for a_ref in (a0_ref, a1_ref, a2_ref, a3_ref, a4_ref)], axis=0)
        r1 = jnp.maximum(h[:, :nhid] + gc1_b_ref[...], 0.0)
        scv = jnp.dot(r1, w2c_ref[...], preferred_element_type=jnp.float32)
        sc_s[rows, :] = scv[:, 0:8].astype(jnp.bfloat16)
        misc_s[rows, 8:16] = h[:, nhid:] + gc2_b_ref[...]
        misc_s[rows, 16:24] = scv[:, 8:16] + misc_s[rows, 0:8]

    @pl.when(j == 1)
    def _pass2():
        hc = jnp.concatenate(
            [jnp.dot(a_ref[...].astype(jnp.bfloat16), sc_s[...],
                     preferred_element_type=jnp.float32)
             for a_ref in (a0_ref, a1_ref, a2_ref, a3_ref, a4_ref)], axis=0)
        out_ref[...] = (jnp.maximum(hc + misc_s[rows, 8:16], 0.0)
                        + misc_s[rows, 16:24])


@jax.jit
def kernel(x, adj, gc1_w, gc1_b, gc2_w, gc2_b,
           lin1_w, lin1_b, lin2_w, lin2_b):
    n, nfeat = x.shape
    nhid = gc1_w.shape[1]
    ncls = gc2_w.shape[1]
    w2c = jnp.concatenate([gc2_w, lin2_w], axis=1)

    full = lambda r, c: pl.BlockSpec((r, c), lambda j, i: (0, 0))

    out = pl.pallas_call(
        _gcn_kernel,
        grid=(2, n // (_NWIN * _BM)),
        in_specs=[
            full(n, nfeat),
            *[pl.BlockSpec((_BM, n), lambda j, i, k=k: (_NWIN * i + k, 0))
              for k in range(_NWIN)],
            full(nfeat, nhid),
            full(1, nhid),
            full(1, ncls),
            full(nfeat, nhid),
            full(1, nhid),
            full(nhid, 2 * ncls),
            full(1, ncls),
        ],
        out_specs=pl.BlockSpec((_NWIN * _BM, ncls), lambda j, i: (i, 0)),
        out_shape=jax.ShapeDtypeStruct((n, ncls), jnp.float32),
        scratch_shapes=[
            pltpu.VMEM((n, nhid + ncls), jnp.bfloat16),
            pltpu.VMEM((n, ncls), jnp.bfloat16),
            pltpu.VMEM((n, 3 * ncls), jnp.float32),
        ],
        compiler_params=pltpu.CompilerParams(
            dimension_semantics=("arbitrary", "arbitrary"),
        ),
    )(x, adj, adj, adj, adj, adj, gc1_w, gc1_b.reshape(1, nhid), gc2_b.reshape(1, ncls),
      lin1_w, lin1_b.reshape(1, nhid), w2c, lin2_b.reshape(1, ncls))
    return out
